# Initial kernel scaffold; baseline (speedup 1.0000x reference)
import jax
import jax.numpy as jnp
from jax.experimental import pallas as pl


def kernel(features, edge_index, W1, b1, W2, b2, Wc, bc):
    raise NotImplementedError("write your pallas kernel here")


# speedup vs baseline: 8.6857x; 8.6857x over previous
import functools

import jax
import jax.numpy as jnp
from jax import lax
from jax.experimental import pallas as pl
from jax.experimental.pallas import tpu as pltpu
from jax.experimental.pallas import tpu_sc as plsc

_LANES = 16
_NC = 2
_NS = 16
_NW = _NC * _NS
_C = 128


def _bf16_rn(x):
    u = lax.bitcast_convert_type(x, jnp.uint32)
    r = u + jnp.uint32(0x7FFF) + ((u >> 16) & jnp.uint32(1))
    return lax.bitcast_convert_type(r & jnp.uint32(0xFFFF0000), jnp.float32)


def _make_edge_agg(n_pad, ch):
    slc = n_pad // _NS
    mesh = plsc.VectorSubcoreMesh(core_axis_name="c", subcore_axis_name="s")

    @functools.partial(
        pl.kernel,
        out_type=jax.ShapeDtypeStruct((_NC * n_pad,), jnp.float32),
        mesh=mesh,
        compiler_params=pltpu.CompilerParams(needs_layout_passes=False),
        scratch_types=[
            pltpu.VMEM((n_pad,), jnp.float32),
            pltpu.VMEM((ch, _C), jnp.int32),
            pltpu.VMEM((ch, _C), jnp.int32),
            pltpu.VMEM((_C,), jnp.float32),
            pltpu.VMEM((slc,), jnp.float32),
            pltpu.VMEM_SHARED((n_pad,), jnp.float32),
        ],
    )
    def agg(values_hbm, src_hbm, dst_hbm, out_hbm,
            values_v, src_v, dst_v, vals_v, zbuf, acc):
        cid = lax.axis_index("c")
        sid = lax.axis_index("s")
        w = sid * _NC + cid

        def zero_body(i, carry):
            zbuf[pl.ds(i * _LANES, _LANES)] = jnp.zeros((_LANES,), jnp.float32)
            return carry

        lax.fori_loop(0, slc // _LANES, zero_body, 0)
        pltpu.sync_copy(zbuf, acc.at[pl.ds(sid * slc, slc)])
        pltpu.sync_copy(values_hbm, values_v)
        pltpu.sync_copy(src_hbm.at[w], src_v)
        pltpu.sync_copy(dst_hbm.at[w], dst_v)
        plsc.subcore_barrier()

        def chunk_body(i, carry):
            for j in range(_C // _LANES):
                idx = src_v[i, pl.ds(j * _LANES, _LANES)]
                vals_v[pl.ds(j * _LANES, _LANES)] = plsc.load_gather(
                    values_v, [idx])
            pltpu.sync_copy(vals_v, acc.at[dst_v.at[i]], add=True)
            return carry

        lax.fori_loop(0, ch, chunk_body, 0)
        plsc.subcore_barrier()
        pltpu.sync_copy(acc.at[pl.ds(sid * slc, slc)], zbuf)
        pltpu.sync_copy(zbuf, out_hbm.at[pl.ds(cid * n_pad + sid * slc, slc)])

    return agg


def _make_row_agg(n_pad, hdim, ch2):
    nh = n_pad // 2
    rpt = nh // _NS
    wb = next(c for c in range(128, 7, -8) if rpt % c == 0)
    nq = rpt // wb
    mesh = plsc.VectorSubcoreMesh(core_axis_name="c", subcore_axis_name="s")

    @functools.partial(
        pl.kernel,
        out_type=jax.ShapeDtypeStruct((n_pad, hdim), jnp.float32),
        mesh=mesh,
        compiler_params=pltpu.CompilerParams(
            needs_layout_passes=False, use_tc_tiling_on_sc=False),
        scratch_types=[
            pltpu.VMEM((128,), jnp.int32),
            pltpu.VMEM((128,), jnp.int32),
            pltpu.VMEM((128, hdim), jnp.float32),
            pltpu.VMEM((wb, hdim), jnp.float32),
            pltpu.VMEM_SHARED((nh + 8, hdim), jnp.float32),
            pltpu.SemaphoreType.DMA,
        ],
    )
    def agg(h_hbm, src_hbm, dst_hbm, out_hbm,
            src_v, idx_v, rows_v, zbuf, acc, sem):
        cid = lax.axis_index("c")
        sid = lax.axis_index("s")
        base = cid * nh

        def zero_body(i, carry):
            for j in range(hdim // _LANES):
                zbuf[i, pl.ds(j * _LANES, _LANES)] = jnp.zeros(
                    (_LANES,), jnp.float32)
            return carry

        lax.fori_loop(0, wb, zero_body, 0)
        for q in range(nq):
            pltpu.sync_copy(zbuf, acc.at[pl.ds(sid * rpt + q * wb, wb)])

        @pl.when(sid == 0)
        def _():
            pltpu.sync_copy(zbuf.at[pl.ds(0, 8)], acc.at[pl.ds(nh, 8)])

        plsc.subcore_barrier()

        def chunk_body(i, carry):
            pltpu.sync_copy(src_hbm.at[sid, i], src_v)
            pltpu.sync_copy(dst_hbm.at[sid, i], idx_v)
            for j in range(_C // _LANES):
                d = idx_v[pl.ds(j * _LANES, _LANES)]
                local = d - base
                ok = (local >= 0) & (local < nh)
                idx_v[pl.ds(j * _LANES, _LANES)] = jnp.where(ok, local, nh)
            pltpu.async_copy(h_hbm.at[src_v], rows_v, sem).wait()
            pltpu.sync_copy(rows_v, acc.at[idx_v], add=True)
            return carry

        lax.fori_loop(0, ch2, chunk_body, 0)
        plsc.subcore_barrier()
        for q in range(nq):
            r0 = sid * rpt + q * wb
            pltpu.sync_copy(acc.at[pl.ds(r0, wb)], zbuf)
            pltpu.sync_copy(zbuf, out_hbm.at[pl.ds(base + r0, wb)])

    return agg


def _dense1_body(n, blk, t2, w1, b1, wc, bc, h_ref, critic_ref, cacc):
    i = pl.program_id(0)

    @pl.when(i == 0)
    def _():
        cacc[0] = 0.0

    t = t2[0] + t2[1]
    h = jnp.maximum(t * w1[...] + b1[...], 0.0)
    h_ref[...] = h
    idx = i * blk + lax.broadcasted_iota(jnp.int32, h.shape, 0)
    hb = _bf16_rn(h) * wc[...]
    cacc[0] += jnp.sum(jnp.where(idx < n, hb, 0.0))

    @pl.when(i == pl.num_programs(0) - 1)
    def _():
        critic_ref[0, 0] = cacc[0] / n + bc[0]


def _final_body(n, blk, a2, w2, b2, g, node_ref, lp_ref,
                macc, seacc, bacc, selacc, lselacc):
    ph = pl.program_id(0)
    i = pl.program_id(1)

    @pl.when((ph == 0) & (i == 0))
    def _():
        macc[0] = -jnp.inf
        bacc[0] = -jnp.inf
        selacc[0] = 2**31 - 1
        seacc[0] = 0.0
        lselacc[0] = 0.0

    p = _bf16_rn(a2[...]) * w2[...]
    l = jnp.sum(p, axis=1, keepdims=True) + b2[0]
    idx = i * blk + lax.broadcasted_iota(jnp.int32, (blk, 1), 0)
    valid = idx < n

    @pl.when(ph == 0)
    def _():
        lm = jnp.where(valid, l, -jnp.inf)
        macc[0] = jnp.maximum(macc[0], jnp.max(lm))
        y = jnp.where(valid, l + g[...], -jnp.inf)
        bmax = jnp.max(y)
        bsel = jnp.min(jnp.where(y == bmax, idx, 2**31 - 1))
        better = bmax > bacc[0]
        equal = bmax == bacc[0]
        selacc[0] = jnp.where(
            better, bsel,
            jnp.where(equal, jnp.minimum(selacc[0], bsel), selacc[0]))
        bacc[0] = jnp.maximum(bacc[0], bmax)

    @pl.when(ph == 1)
    def _():
        seacc[0] += jnp.sum(jnp.where(valid, jnp.exp(l - macc[0]), 0.0))
        lselacc[0] += jnp.sum(jnp.where(idx == selacc[0], l, 0.0))

    @pl.when((ph == 1) & (i == pl.num_programs(1) - 1))
    def _():
        node_ref[0, 0] = selacc[0]
        lp_ref[0, 0] = (lselacc[0] - macc[0]) - jnp.log(seacc[0])


def _edge_aggregate(values_pad, src_p, dst_p, n_pad, ch):
    flat = _make_edge_agg(n_pad, ch)(values_pad, src_p, dst_p)
    return flat.reshape(_NC, n_pad)


def _row_aggregate(h_arr, src_q, dst_q, n_pad, hdim, ch2):
    return _make_row_agg(n_pad, hdim, ch2)(h_arr, src_q, dst_q)


def kernel(features, edge_index, W1, b1, W2, b2, Wc, bc):
    n = features.shape[0]
    e = edge_index.shape[1]
    hdim = W1.shape[1]
    n_pad = (n // 256 + 1) * 256
    src = edge_index[0]
    dst = edge_index[1]

    ch = -(-e // (_NW * _C))
    e_pad = _NW * _C * ch
    src_p = jnp.concatenate(
        [src, jnp.zeros((e_pad - e,), jnp.int32)]).reshape(_NW, ch, _C)
    dst_p = jnp.concatenate(
        [dst, jnp.full((e_pad - e,), n, jnp.int32)]).reshape(_NW, ch, _C)
    vals0 = jnp.concatenate(
        [features[:, 0], jnp.zeros((n_pad - n,), jnp.float32)])
    t2 = _edge_aggregate(vals0, src_p, dst_p, n_pad, ch).reshape(2, n_pad, 1)

    smem = pl.BlockSpec(memory_space=pltpu.SMEM)
    nb = 8
    blk = n_pad // nb
    h_arr, critic = pl.pallas_call(
        functools.partial(_dense1_body, n, blk),
        grid=(nb,),
        out_shape=(jax.ShapeDtypeStruct((n_pad, hdim), jnp.float32),
                   jax.ShapeDtypeStruct((1, 1), jnp.float32)),
        in_specs=[pl.BlockSpec((2, blk, 1), lambda i: (0, i, 0)),
                  pl.BlockSpec((1, hdim), lambda i: (0, 0)),
                  pl.BlockSpec((1, hdim), lambda i: (0, 0)),
                  pl.BlockSpec((1, hdim), lambda i: (0, 0)),
                  smem],
        out_specs=(pl.BlockSpec((blk, hdim), lambda i: (i, 0)),
                   pl.BlockSpec((1, 1), lambda i: (0, 0),
                                memory_space=pltpu.SMEM)),
        scratch_shapes=[pltpu.SMEM((1,), jnp.float32)],
    )(t2, W1, b1.reshape(1, hdim), _bf16_rn(Wc).reshape(1, hdim), bc)

    ch2 = -(-e // (_NS * _C))
    e_pad2 = _NS * _C * ch2
    src_q = jnp.concatenate(
        [src, jnp.zeros((e_pad2 - e,), jnp.int32)]).reshape(_NS, ch2, _C)
    dst_q = jnp.concatenate(
        [dst, jnp.full((e_pad2 - e,), n, jnp.int32)]).reshape(_NS, ch2, _C)
    agg2 = _row_aggregate(h_arr, src_q, dst_q, n_pad, hdim, ch2)

    u = jax.random.uniform(jax.random.key(42), (n,), minval=1e-9, maxval=1.0)
    g = -jnp.log(-jnp.log(u))
    g_col = jnp.concatenate(
        [g, jnp.zeros((n_pad - n,), jnp.float32)]).reshape(n_pad, 1)
    node, lp = pl.pallas_call(
        functools.partial(_final_body, n, blk),
        grid=(2, nb),
        out_shape=(jax.ShapeDtypeStruct((1, 1), jnp.int32),
                   jax.ShapeDtypeStruct((1, 1), jnp.float32)),
        in_specs=[pl.BlockSpec((blk, hdim), lambda ph, i: (i, 0)),
                  pl.BlockSpec((1, hdim), lambda ph, i: (0, 0)),
                  smem,
                  pl.BlockSpec((blk, 1), lambda ph, i: (i, 0))],
        out_specs=(pl.BlockSpec((1, 1), lambda ph, i: (0, 0),
                                memory_space=pltpu.SMEM),
                   pl.BlockSpec((1, 1), lambda ph, i: (0, 0),
                                memory_space=pltpu.SMEM)),
        scratch_shapes=[pltpu.SMEM((1,), jnp.float32),
                        pltpu.SMEM((1,), jnp.float32),
                        pltpu.SMEM((1,), jnp.float32),
                        pltpu.SMEM((1,), jnp.int32),
                        pltpu.SMEM((1,), jnp.float32)],
    )(agg2, _bf16_rn(W2).reshape(1, hdim), b2, g_col)

    return node.reshape(()), lp.reshape(()), critic.reshape(())


# speedup vs baseline: 11.4675x; 1.3203x over previous
import functools

import jax
import jax.numpy as jnp
from jax import lax
from jax.experimental import pallas as pl
from jax.experimental.pallas import tpu as pltpu
from jax.experimental.pallas import tpu_sc as plsc

_LANES = 16
_NC = 2
_NS = 16
_NW = _NC * _NS
_C = 128


def _bf16_rn(x):
    u = lax.bitcast_convert_type(x, jnp.uint32)
    r = u + jnp.uint32(0x7FFF) + ((u >> 16) & jnp.uint32(1))
    return lax.bitcast_convert_type(r & jnp.uint32(0xFFFF0000), jnp.float32)


def _make_edge_agg(n_pad, ch):
    slc = n_pad // _NS
    mesh = plsc.VectorSubcoreMesh(core_axis_name="c", subcore_axis_name="s")

    @functools.partial(
        pl.kernel,
        out_type=jax.ShapeDtypeStruct((_NC * n_pad,), jnp.float32),
        mesh=mesh,
        compiler_params=pltpu.CompilerParams(needs_layout_passes=False),
        scratch_types=[
            pltpu.VMEM((n_pad,), jnp.float32),
            pltpu.VMEM((ch, _C), jnp.int32),
            pltpu.VMEM((ch, _C), jnp.int32),
            pltpu.VMEM((_C,), jnp.float32),
            pltpu.VMEM((slc,), jnp.float32),
            pltpu.VMEM_SHARED((n_pad,), jnp.float32),
        ],
    )
    def agg(values_hbm, src_hbm, dst_hbm, out_hbm,
            values_v, src_v, dst_v, vals_v, zbuf, acc):
        cid = lax.axis_index("c")
        sid = lax.axis_index("s")
        w = sid * _NC + cid

        def zero_body(i, carry):
            zbuf[pl.ds(i * _LANES, _LANES)] = jnp.zeros((_LANES,), jnp.float32)
            return carry

        lax.fori_loop(0, slc // _LANES, zero_body, 0)
        pltpu.sync_copy(zbuf, acc.at[pl.ds(sid * slc, slc)])
        pltpu.sync_copy(values_hbm, values_v)
        pltpu.sync_copy(src_hbm.at[w], src_v)
        pltpu.sync_copy(dst_hbm.at[w], dst_v)
        plsc.subcore_barrier()

        def chunk_body(i, carry):
            for j in range(_C // _LANES):
                idx = src_v[i, pl.ds(j * _LANES, _LANES)]
                vals_v[pl.ds(j * _LANES, _LANES)] = plsc.load_gather(
                    values_v, [idx])
            pltpu.sync_copy(vals_v, acc.at[dst_v.at[i]], add=True)
            return carry

        lax.fori_loop(0, ch, chunk_body, 0)
        plsc.subcore_barrier()
        pltpu.sync_copy(acc.at[pl.ds(sid * slc, slc)], zbuf)
        pltpu.sync_copy(zbuf, out_hbm.at[pl.ds(cid * n_pad + sid * slc, slc)])

    return agg


def _make_row_agg(n_pad, hdim, ch2):
    nh = n_pad // 2
    rpt = nh // _NS
    wb = next(c for c in range(56, 7, -8) if rpt % c == 0)
    nq = rpt // wb
    bch = next(c for c in range(28, 1, -2) if ch2 % c == 0)
    nbatch = ch2 // bch
    mesh = plsc.VectorSubcoreMesh(core_axis_name="c", subcore_axis_name="s")

    @functools.partial(
        pl.kernel,
        out_type=jax.ShapeDtypeStruct((n_pad, hdim), jnp.float32),
        mesh=mesh,
        compiler_params=pltpu.CompilerParams(
            needs_layout_passes=False, use_tc_tiling_on_sc=False),
        scratch_types=[
            pltpu.VMEM((bch, _C), jnp.int32),
            pltpu.VMEM((bch, _C), jnp.int32),
            pltpu.VMEM((_C, hdim), jnp.float32),
            pltpu.VMEM((_C, hdim), jnp.float32),
            pltpu.VMEM((wb, hdim), jnp.float32),
            pltpu.VMEM_SHARED((nh + 8, hdim), jnp.float32),
            pltpu.SemaphoreType.DMA,
            pltpu.SemaphoreType.DMA,
            pltpu.SemaphoreType.DMA,
            pltpu.SemaphoreType.DMA,
        ],
    )
    def agg(h_hbm, src_hbm, dst_hbm, out_hbm,
            src_v, idx_v, rows0, rows1, zbuf, acc, g0, g1, s0, s1):
        cid = lax.axis_index("c")
        sid = lax.axis_index("s")
        base = cid * nh

        def zero_body(i, carry):
            for j in range(hdim // _LANES):
                zbuf[i, pl.ds(j * _LANES, _LANES)] = jnp.zeros(
                    (_LANES,), jnp.float32)
            return carry

        lax.fori_loop(0, wb, zero_body, 0)
        for q in range(nq):
            pltpu.sync_copy(zbuf, acc.at[pl.ds(sid * rpt + q * wb, wb)])

        @pl.when(sid == 0)
        def _():
            pltpu.sync_copy(zbuf.at[pl.ds(0, 8)], acc.at[pl.ds(nh, 8)])

        plsc.subcore_barrier()

        bufs = ((rows0, g0, s0), (rows1, g1, s1))

        def batch_body(b, carry):
            pltpu.sync_copy(src_hbm.at[sid, pl.ds(b * bch, bch)], src_v)
            pltpu.sync_copy(dst_hbm.at[sid, pl.ds(b * bch, bch)], idx_v)

            def adj_body(i, c2):
                for j in range(_C // _LANES):
                    d = idx_v[i, pl.ds(j * _LANES, _LANES)]
                    local = d - base
                    ok = (local >= 0) & (local < nh)
                    idx_v[i, pl.ds(j * _LANES, _LANES)] = jnp.where(
                        ok, local, nh)
                return c2

            lax.fori_loop(0, bch, adj_body, 0)
            pltpu.async_copy(h_hbm.at[src_v.at[0]], rows0, g0)
            pltpu.async_copy(h_hbm.at[src_v.at[1]], rows1, g1)
            for k in range(bch):
                buf, gs, ss = bufs[k % 2]
                pltpu.make_async_copy(h_hbm.at[src_v.at[k]], buf, gs).wait()
                pltpu.async_copy(buf, acc.at[idx_v.at[k]], ss, add=True)
                pltpu.make_async_copy(buf, acc.at[idx_v.at[k]], ss).wait()
                if k + 2 < bch:
                    pltpu.async_copy(h_hbm.at[src_v.at[k + 2]], buf, gs)
            return carry

        lax.fori_loop(0, nbatch, batch_body, 0)
        plsc.subcore_barrier()
        for q in range(nq):
            r0 = sid * rpt + q * wb
            pltpu.sync_copy(acc.at[pl.ds(r0, wb)], zbuf)
            pltpu.sync_copy(zbuf, out_hbm.at[pl.ds(base + r0, wb)])

    return agg


def _dense1_body(n, blk, t2, w1, b1, wc, bc, h_ref, critic_ref, cacc):
    i = pl.program_id(0)

    @pl.when(i == 0)
    def _():
        cacc[0] = 0.0

    t = t2[0] + t2[1]
    h = jnp.maximum(t * w1[...] + b1[...], 0.0)
    h_ref[...] = h
    idx = i * blk + lax.broadcasted_iota(jnp.int32, h.shape, 0)
    hb = _bf16_rn(h) * wc[...]
    cacc[0] += jnp.sum(jnp.where(idx < n, hb, 0.0))

    @pl.when(i == pl.num_programs(0) - 1)
    def _():
        critic_ref[0, 0] = cacc[0] / n + bc[0]


def _final_body(n, blk, a2, w2, b2, g, node_ref, lp_ref,
                macc, seacc, bacc, selacc, lselacc):
    ph = pl.program_id(0)
    i = pl.program_id(1)

    @pl.when((ph == 0) & (i == 0))
    def _():
        macc[0] = -jnp.inf
        bacc[0] = -jnp.inf
        selacc[0] = 2**31 - 1
        seacc[0] = 0.0
        lselacc[0] = 0.0

    p = _bf16_rn(a2[...]) * w2[...]
    l = jnp.sum(p, axis=1, keepdims=True) + b2[0]
    idx = i * blk + lax.broadcasted_iota(jnp.int32, (blk, 1), 0)
    valid = idx < n

    @pl.when(ph == 0)
    def _():
        lm = jnp.where(valid, l, -jnp.inf)
        macc[0] = jnp.maximum(macc[0], jnp.max(lm))
        y = jnp.where(valid, l + g[...], -jnp.inf)
        bmax = jnp.max(y)
        bsel = jnp.min(jnp.where(y == bmax, idx, 2**31 - 1))
        better = bmax > bacc[0]
        equal = bmax == bacc[0]
        selacc[0] = jnp.where(
            better, bsel,
            jnp.where(equal, jnp.minimum(selacc[0], bsel), selacc[0]))
        bacc[0] = jnp.maximum(bacc[0], bmax)

    @pl.when(ph == 1)
    def _():
        seacc[0] += jnp.sum(jnp.where(valid, jnp.exp(l - macc[0]), 0.0))
        lselacc[0] += jnp.sum(jnp.where(idx == selacc[0], l, 0.0))

    @pl.when((ph == 1) & (i == pl.num_programs(1) - 1))
    def _():
        node_ref[0, 0] = selacc[0]
        lp_ref[0, 0] = (lselacc[0] - macc[0]) - jnp.log(seacc[0])


def _edge_aggregate(values_pad, src_p, dst_p, n_pad, ch):
    flat = _make_edge_agg(n_pad, ch)(values_pad, src_p, dst_p)
    return flat.reshape(_NC, n_pad)


def _row_aggregate(h_arr, src_q, dst_q, n_pad, hdim, ch2):
    return _make_row_agg(n_pad, hdim, ch2)(h_arr, src_q, dst_q)


def kernel(features, edge_index, W1, b1, W2, b2, Wc, bc):
    n = features.shape[0]
    e = edge_index.shape[1]
    hdim = W1.shape[1]
    n_pad = (n // 256 + 1) * 256
    src = edge_index[0]
    dst = edge_index[1]

    ch = -(-e // (_NW * _C))
    e_pad = _NW * _C * ch
    src_p = jnp.concatenate(
        [src, jnp.zeros((e_pad - e,), jnp.int32)]).reshape(_NW, ch, _C)
    dst_p = jnp.concatenate(
        [dst, jnp.full((e_pad - e,), n, jnp.int32)]).reshape(_NW, ch, _C)
    vals0 = jnp.concatenate(
        [features[:, 0], jnp.zeros((n_pad - n,), jnp.float32)])
    t2 = _edge_aggregate(vals0, src_p, dst_p, n_pad, ch).reshape(2, n_pad, 1)

    smem = pl.BlockSpec(memory_space=pltpu.SMEM)
    nb = 8
    blk = n_pad // nb
    h_arr, critic = pl.pallas_call(
        functools.partial(_dense1_body, n, blk),
        grid=(nb,),
        out_shape=(jax.ShapeDtypeStruct((n_pad, hdim), jnp.float32),
                   jax.ShapeDtypeStruct((1, 1), jnp.float32)),
        in_specs=[pl.BlockSpec((2, blk, 1), lambda i: (0, i, 0)),
                  pl.BlockSpec((1, hdim), lambda i: (0, 0)),
                  pl.BlockSpec((1, hdim), lambda i: (0, 0)),
                  pl.BlockSpec((1, hdim), lambda i: (0, 0)),
                  smem],
        out_specs=(pl.BlockSpec((blk, hdim), lambda i: (i, 0)),
                   pl.BlockSpec((1, 1), lambda i: (0, 0),
                                memory_space=pltpu.SMEM)),
        scratch_shapes=[pltpu.SMEM((1,), jnp.float32)],
    )(t2, W1, b1.reshape(1, hdim), _bf16_rn(Wc).reshape(1, hdim), bc)

    ch2 = -(-e // (_NS * _C))
    ch2 = ch2 + (ch2 % 2)
    e_pad2 = _NS * _C * ch2
    src_q = jnp.concatenate(
        [src, jnp.zeros((e_pad2 - e,), jnp.int32)]).reshape(_NS, ch2, _C)
    dst_q = jnp.concatenate(
        [dst, jnp.full((e_pad2 - e,), n, jnp.int32)]).reshape(_NS, ch2, _C)
    agg2 = _row_aggregate(h_arr, src_q, dst_q, n_pad, hdim, ch2)

    u = jax.random.uniform(jax.random.key(42), (n,), minval=1e-9, maxval=1.0)
    g = -jnp.log(-jnp.log(u))
    g_col = jnp.concatenate(
        [g, jnp.zeros((n_pad - n,), jnp.float32)]).reshape(n_pad, 1)
    node, lp = pl.pallas_call(
        functools.partial(_final_body, n, blk),
        grid=(2, nb),
        out_shape=(jax.ShapeDtypeStruct((1, 1), jnp.int32),
                   jax.ShapeDtypeStruct((1, 1), jnp.float32)),
        in_specs=[pl.BlockSpec((blk, hdim), lambda ph, i: (i, 0)),
                  pl.BlockSpec((1, hdim), lambda ph, i: (0, 0)),
                  smem,
                  pl.BlockSpec((blk, 1), lambda ph, i: (i, 0))],
        out_specs=(pl.BlockSpec((1, 1), lambda ph, i: (0, 0),
                                memory_space=pltpu.SMEM),
                   pl.BlockSpec((1, 1), lambda ph, i: (0, 0),
                                memory_space=pltpu.SMEM)),
        scratch_shapes=[pltpu.SMEM((1,), jnp.float32),
                        pltpu.SMEM((1,), jnp.float32),
                        pltpu.SMEM((1,), jnp.float32),
                        pltpu.SMEM((1,), jnp.int32),
                        pltpu.SMEM((1,), jnp.float32)],
    )(agg2, _bf16_rn(W2).reshape(1, hdim), b2, g_col)

    return node.reshape(()), lp.reshape(()), critic.reshape(())


# speedup vs baseline: 14.7355x; 1.2850x over previous
import functools

import jax
import jax.numpy as jnp
from jax import lax
from jax.experimental import pallas as pl
from jax.experimental.pallas import tpu as pltpu
from jax.experimental.pallas import tpu_sc as plsc

_LANES = 16
_NC = 2
_NS = 16
_NW = _NC * _NS
_C = 128


def _bf16_rn(x):
    u = lax.bitcast_convert_type(x, jnp.uint32)
    r = u + jnp.uint32(0x7FFF) + ((u >> 16) & jnp.uint32(1))
    return lax.bitcast_convert_type(r & jnp.uint32(0xFFFF0000), jnp.float32)


def _make_edge_agg(n_pad, ch):
    slc = n_pad // _NS
    mesh = plsc.VectorSubcoreMesh(core_axis_name="c", subcore_axis_name="s")

    @functools.partial(
        pl.kernel,
        out_type=jax.ShapeDtypeStruct((_NC * n_pad,), jnp.float32),
        mesh=mesh,
        compiler_params=pltpu.CompilerParams(needs_layout_passes=False),
        scratch_types=[
            pltpu.VMEM((n_pad,), jnp.float32),
            pltpu.VMEM((ch, _C), jnp.int32),
            pltpu.VMEM((ch, _C), jnp.int32),
            pltpu.VMEM((_C,), jnp.float32),
            pltpu.VMEM((slc,), jnp.float32),
            pltpu.VMEM_SHARED((n_pad,), jnp.float32),
        ],
    )
    def agg(values_hbm, src_hbm, dst_hbm, out_hbm,
            values_v, src_v, dst_v, vals_v, zbuf, acc):
        cid = lax.axis_index("c")
        sid = lax.axis_index("s")
        w = sid * _NC + cid

        def zero_body(i, carry):
            zbuf[pl.ds(i * _LANES, _LANES)] = jnp.zeros((_LANES,), jnp.float32)
            return carry

        lax.fori_loop(0, slc // _LANES, zero_body, 0)
        pltpu.sync_copy(zbuf, acc.at[pl.ds(sid * slc, slc)])
        pltpu.sync_copy(values_hbm, values_v)
        pltpu.sync_copy(src_hbm.at[w], src_v)
        pltpu.sync_copy(dst_hbm.at[w], dst_v)
        plsc.subcore_barrier()

        def chunk_body(i, carry):
            for j in range(_C // _LANES):
                idx = src_v[i, pl.ds(j * _LANES, _LANES)]
                vals_v[pl.ds(j * _LANES, _LANES)] = plsc.load_gather(
                    values_v, [idx])
            pltpu.sync_copy(vals_v, acc.at[dst_v.at[i]], add=True)
            return carry

        lax.fori_loop(0, ch, chunk_body, 0)
        plsc.subcore_barrier()
        pltpu.sync_copy(acc.at[pl.ds(sid * slc, slc)], zbuf)
        pltpu.sync_copy(zbuf, out_hbm.at[pl.ds(cid * n_pad + sid * slc, slc)])

    return agg


def _make_row_agg(n_pad, hdim, ch2):
    nh = n_pad // 2
    rpt = nh // _NS
    wb = next(c for c in range(56, 7, -8) if rpt % c == 0)
    nq = rpt // wb
    bch = next(c for c in range(28, 1, -2) if ch2 % c == 0)
    nbatch = ch2 // bch
    mesh = plsc.VectorSubcoreMesh(core_axis_name="c", subcore_axis_name="s")

    @functools.partial(
        pl.kernel,
        out_type=jax.ShapeDtypeStruct((n_pad, hdim), jnp.float32),
        mesh=mesh,
        compiler_params=pltpu.CompilerParams(
            needs_layout_passes=False, use_tc_tiling_on_sc=False),
        scratch_types=[
            pltpu.VMEM((bch, _C), jnp.int32),
            pltpu.VMEM((bch, _C), jnp.int32),
            pltpu.VMEM((_C, hdim), jnp.float32),
            pltpu.VMEM((_C, hdim), jnp.float32),
            pltpu.VMEM((wb, hdim), jnp.float32),
            pltpu.VMEM_SHARED((nh + 16, hdim), jnp.float32),
            pltpu.SemaphoreType.DMA,
            pltpu.SemaphoreType.DMA,
            pltpu.SemaphoreType.DMA,
            pltpu.SemaphoreType.DMA,
        ],
    )
    def agg(h_hbm, src_hbm, dst_hbm, out_hbm,
            src_v, idx_v, rows0, rows1, zbuf, acc, g0, g1, s0, s1):
        cid = lax.axis_index("c")
        sid = lax.axis_index("s")
        base = cid * nh

        def zero_body(i, carry):
            for j in range(hdim // _LANES):
                zbuf[i, pl.ds(j * _LANES, _LANES)] = jnp.zeros(
                    (_LANES,), jnp.float32)
            return carry

        lax.fori_loop(0, wb, zero_body, 0)
        for q in range(nq):
            pltpu.sync_copy(zbuf, acc.at[pl.ds(sid * rpt + q * wb, wb)])

        @pl.when(sid == 0)
        def _():
            pltpu.sync_copy(zbuf.at[pl.ds(0, 16)], acc.at[pl.ds(nh, 16)])

        plsc.subcore_barrier()

        bufs = ((rows0, g0, s0), (rows1, g1, s1))

        def batch_body(b, carry):
            pltpu.sync_copy(src_hbm.at[sid, pl.ds(b * bch, bch)], src_v)
            pltpu.sync_copy(dst_hbm.at[sid, pl.ds(b * bch, bch)], idx_v)

            dump = nh + lax.iota(jnp.int32, 16)

            def adj_body(i, c2):
                for j in range(_C // _LANES):
                    d = idx_v[i, pl.ds(j * _LANES, _LANES)]
                    local = d - base
                    ok = (local >= 0) & (local < nh)
                    idx_v[i, pl.ds(j * _LANES, _LANES)] = jnp.where(
                        ok, local, dump)
                return c2

            lax.fori_loop(0, bch, adj_body, 0)
            pltpu.async_copy(h_hbm.at[src_v.at[0]], rows0, g0)
            pltpu.async_copy(h_hbm.at[src_v.at[1]], rows1, g1)
            for k in range(bch):
                buf, gs, ss = bufs[k % 2]
                pltpu.make_async_copy(h_hbm.at[src_v.at[k]], buf, gs).wait()
                pltpu.async_copy(buf, acc.at[idx_v.at[k]], ss, add=True)
                pltpu.make_async_copy(buf, acc.at[idx_v.at[k]], ss).wait()
                if k + 2 < bch:
                    pltpu.async_copy(h_hbm.at[src_v.at[k + 2]], buf, gs)
            return carry

        lax.fori_loop(0, nbatch, batch_body, 0)
        plsc.subcore_barrier()
        for q in range(nq):
            r0 = sid * rpt + q * wb
            pltpu.sync_copy(acc.at[pl.ds(r0, wb)], zbuf)
            pltpu.sync_copy(zbuf, out_hbm.at[pl.ds(base + r0, wb)])

    return agg


def _dense1_body(n, blk, t2, w1, b1, wc, bc, h_ref, critic_ref, cacc):
    i = pl.program_id(0)

    @pl.when(i == 0)
    def _():
        cacc[0] = 0.0

    t = t2[0] + t2[1]
    h = jnp.maximum(t * w1[...] + b1[...], 0.0)
    h_ref[...] = h
    idx = i * blk + lax.broadcasted_iota(jnp.int32, h.shape, 0)
    hb = _bf16_rn(h) * wc[...]
    cacc[0] += jnp.sum(jnp.where(idx < n, hb, 0.0))

    @pl.when(i == pl.num_programs(0) - 1)
    def _():
        critic_ref[0, 0] = cacc[0] / n + bc[0]


def _final_body(n, blk, a2, w2, b2, g, node_ref, lp_ref,
                macc, seacc, bacc, selacc, lselacc):
    ph = pl.program_id(0)
    i = pl.program_id(1)

    @pl.when((ph == 0) & (i == 0))
    def _():
        macc[0] = -jnp.inf
        bacc[0] = -jnp.inf
        selacc[0] = 2**31 - 1
        seacc[0] = 0.0
        lselacc[0] = 0.0

    p = _bf16_rn(a2[...]) * w2[...]
    l = jnp.sum(p, axis=1, keepdims=True) + b2[0]
    idx = i * blk + lax.broadcasted_iota(jnp.int32, (blk, 1), 0)
    valid = idx < n

    @pl.when(ph == 0)
    def _():
        lm = jnp.where(valid, l, -jnp.inf)
        macc[0] = jnp.maximum(macc[0], jnp.max(lm))
        y = jnp.where(valid, l + g[...], -jnp.inf)
        bmax = jnp.max(y)
        bsel = jnp.min(jnp.where(y == bmax, idx, 2**31 - 1))
        better = bmax > bacc[0]
        equal = bmax == bacc[0]
        selacc[0] = jnp.where(
            better, bsel,
            jnp.where(equal, jnp.minimum(selacc[0], bsel), selacc[0]))
        bacc[0] = jnp.maximum(bacc[0], bmax)

    @pl.when(ph == 1)
    def _():
        seacc[0] += jnp.sum(jnp.where(valid, jnp.exp(l - macc[0]), 0.0))
        lselacc[0] += jnp.sum(jnp.where(idx == selacc[0], l, 0.0))

    @pl.when((ph == 1) & (i == pl.num_programs(1) - 1))
    def _():
        node_ref[0, 0] = selacc[0]
        lp_ref[0, 0] = (lselacc[0] - macc[0]) - jnp.log(seacc[0])


def _edge_aggregate(values_pad, src_p, dst_p, n_pad, ch):
    flat = _make_edge_agg(n_pad, ch)(values_pad, src_p, dst_p)
    return flat.reshape(_NC, n_pad)


def _row_aggregate(h_arr, src_q, dst_q, n_pad, hdim, ch2):
    return _make_row_agg(n_pad, hdim, ch2)(h_arr, src_q, dst_q)


def kernel(features, edge_index, W1, b1, W2, b2, Wc, bc):
    n = features.shape[0]
    e = edge_index.shape[1]
    hdim = W1.shape[1]
    n_pad = (n // 256 + 1) * 256
    src = edge_index[0]
    dst = edge_index[1]

    ch = -(-e // (_NW * _C))
    e_pad = _NW * _C * ch
    src_p = jnp.concatenate(
        [src, jnp.zeros((e_pad - e,), jnp.int32)]).reshape(_NW, ch, _C)
    dst_p = jnp.concatenate(
        [dst, jnp.full((e_pad - e,), n, jnp.int32)]).reshape(_NW, ch, _C)
    vals0 = jnp.concatenate(
        [features[:, 0], jnp.zeros((n_pad - n,), jnp.float32)])
    t2 = _edge_aggregate(vals0, src_p, dst_p, n_pad, ch).reshape(2, n_pad, 1)

    smem = pl.BlockSpec(memory_space=pltpu.SMEM)
    nb = 8
    blk = n_pad // nb
    h_arr, critic = pl.pallas_call(
        functools.partial(_dense1_body, n, blk),
        grid=(nb,),
        out_shape=(jax.ShapeDtypeStruct((n_pad, hdim), jnp.float32),
                   jax.ShapeDtypeStruct((1, 1), jnp.float32)),
        in_specs=[pl.BlockSpec((2, blk, 1), lambda i: (0, i, 0)),
                  pl.BlockSpec((1, hdim), lambda i: (0, 0)),
                  pl.BlockSpec((1, hdim), lambda i: (0, 0)),
                  pl.BlockSpec((1, hdim), lambda i: (0, 0)),
                  smem],
        out_specs=(pl.BlockSpec((blk, hdim), lambda i: (i, 0)),
                   pl.BlockSpec((1, 1), lambda i: (0, 0),
                                memory_space=pltpu.SMEM)),
        scratch_shapes=[pltpu.SMEM((1,), jnp.float32)],
    )(t2, W1, b1.reshape(1, hdim), _bf16_rn(Wc).reshape(1, hdim), bc)

    ch2 = -(-e // (_NS * _C))
    ch2 = ch2 + (ch2 % 2)
    e_pad2 = _NS * _C * ch2
    src_q = jnp.concatenate(
        [src, jnp.zeros((e_pad2 - e,), jnp.int32)]).reshape(_NS, ch2, _C)
    dst_q = jnp.concatenate(
        [dst, jnp.full((e_pad2 - e,), n, jnp.int32)]).reshape(_NS, ch2, _C)
    agg2 = _row_aggregate(h_arr, src_q, dst_q, n_pad, hdim, ch2)

    u = jax.random.uniform(jax.random.key(42), (n,), minval=1e-9, maxval=1.0)
    g = -jnp.log(-jnp.log(u))
    g_col = jnp.concatenate(
        [g, jnp.zeros((n_pad - n,), jnp.float32)]).reshape(n_pad, 1)
    node, lp = pl.pallas_call(
        functools.partial(_final_body, n, blk),
        grid=(2, nb),
        out_shape=(jax.ShapeDtypeStruct((1, 1), jnp.int32),
                   jax.ShapeDtypeStruct((1, 1), jnp.float32)),
        in_specs=[pl.BlockSpec((blk, hdim), lambda ph, i: (i, 0)),
                  pl.BlockSpec((1, hdim), lambda ph, i: (0, 0)),
                  smem,
                  pl.BlockSpec((blk, 1), lambda ph, i: (i, 0))],
        out_specs=(pl.BlockSpec((1, 1), lambda ph, i: (0, 0),
                                memory_space=pltpu.SMEM),
                   pl.BlockSpec((1, 1), lambda ph, i: (0, 0),
                                memory_space=pltpu.SMEM)),
        scratch_shapes=[pltpu.SMEM((1,), jnp.float32),
                        pltpu.SMEM((1,), jnp.float32),
                        pltpu.SMEM((1,), jnp.float32),
                        pltpu.SMEM((1,), jnp.int32),
                        pltpu.SMEM((1,), jnp.float32)],
    )(agg2, _bf16_rn(W2).reshape(1, hdim), b2, g_col)

    return node.reshape(()), lp.reshape(()), critic.reshape(())
